# baseline (speedup 1.0000x reference)
import functools
import jax
import jax.numpy as jnp
from jax import lax
from jax.experimental import pallas as pl
from jax.experimental.pallas import tpu as pltpu
from jax.experimental.pallas import tpu_sc as plsc

NC, NS, L = 2, 16, 16
NW = NC * NS
V, D = 1000, 128
DI = D // 2
NG = D // 32


def kernel(x_train_char, char_emb_weight):
    B, Lw, C = x_train_char.shape
    W = B * Lw
    wpb = W // NW
    CH = 200
    nchunk = wpb // CH

    wp = char_emb_weight.astype(jnp.bfloat16)
    wp = wp.reshape(V, NG, 2, 16).transpose(0, 1, 3, 2).reshape(V, DI, 2)
    tbl = lax.bitcast_convert_type(wp, jnp.int32).reshape(V * DI)

    idx = x_train_char.reshape(W * C)

    @functools.partial(
        pl.kernel,
        out_type=jax.ShapeDtypeStruct((W * D,), jnp.float32),
        mesh=plsc.VectorSubcoreMesh(
            core_axis_name="c", subcore_axis_name="s",
            num_cores=NC, num_subcores=NS,
        ),
        scratch_types=[
            pltpu.VMEM((V * DI,), jnp.int32),
            pltpu.VMEM((CH * C,), jnp.int32),
            pltpu.VMEM((CH * C,), jnp.int32),
            pltpu.VMEM((CH * D,), jnp.float32),
            pltpu.VMEM((CH * D,), jnp.float32),
            pltpu.SemaphoreType.DMA,
            pltpu.SemaphoreType.DMA,
            pltpu.SemaphoreType.DMA,
            pltpu.SemaphoreType.DMA,
        ],
        compiler_params=pltpu.CompilerParams(needs_layout_passes=False),
    )
    def sc_kernel(idx_hbm, tbl_hbm, out_hbm,
                  tbl_v, idx0, idx1, out0, out1,
                  isem0, isem1, osem0, osem1):
        wid = lax.axis_index("s") * NC + lax.axis_index("c")
        base_w = wid * wpb
        pltpu.async_copy(
            idx_hbm.at[pl.ds(base_w * C, CH * C)], idx0, isem0)
        pltpu.sync_copy(tbl_hbm, tbl_v)

        def do_word(idx_v, out_v, w):
            acc0 = [jnp.zeros((2 * L,), jnp.bfloat16) for _ in range(NG)]
            acc1 = [jnp.zeros((2 * L,), jnp.bfloat16) for _ in range(NG)]
            p = w * C
            iv0 = idx_v[pl.ds(p, L)]
            iv1 = idx_v[pl.ds(p + C - L, L)]
            for c in range(C):
                row = iv0[c] if c < L else iv1[c - (C - L)]
                rb = row * DI
                accs = acc0 if c % 2 == 0 else acc1
                for j in range(NG):
                    vals = tbl_v[pl.ds(rb + j * L, L)]
                    accs[j] = accs[j] + plsc.bitcast(vals, jnp.bfloat16)
            ob = w * D
            for j in range(NG):
                ai = plsc.bitcast(acc0[j] + acc1[j], jnp.int32)
                lo = plsc.bitcast(ai << 16, jnp.float32)
                hi = plsc.bitcast(ai & jnp.int32(-65536), jnp.float32)
                out_v[pl.ds(ob + 32 * j, L)] = lo
                out_v[pl.ds(ob + 32 * j + 16, L)] = hi

        def compute_chunk(idx_v, out_v):
            @plsc.parallel_loop(0, CH, unroll=5)
            def word_body(w):
                do_word(idx_v, out_v, w)

        def half(g, idx_v, out_v, isem_this, isem_next, osem, idx_next):
            pltpu.make_async_copy(
                idx_hbm.at[pl.ds(0, CH * C)], idx_v, isem_this).wait()
            gn = jnp.minimum(g + 1, nchunk - 1)
            pltpu.async_copy(
                idx_hbm.at[pl.ds((base_w + gn * CH) * C, CH * C)],
                idx_next, isem_next)

            @pl.when(g >= 2)
            def _():
                pltpu.make_async_copy(
                    out_v, out_hbm.at[pl.ds(0, CH * D)], osem).wait()

            compute_chunk(idx_v, out_v)
            pltpu.async_copy(
                out_v, out_hbm.at[pl.ds((base_w + g * CH) * D, CH * D)], osem)

        def body(g2, _):
            g = g2 * 2
            half(g, idx0, out0, isem0, isem1, osem0, idx1)
            half(g + 1, idx1, out1, isem1, isem0, osem1, idx0)
            return None

        lax.fori_loop(0, nchunk // 2, body, None)

        pltpu.make_async_copy(
            idx_hbm.at[pl.ds(0, CH * C)], idx0, isem0).wait()
        pltpu.make_async_copy(
            out0, out_hbm.at[pl.ds(0, CH * D)], osem0).wait()
        pltpu.make_async_copy(
            out1, out_hbm.at[pl.ds(0, CH * D)], osem1).wait()

    out = sc_kernel(idx, tbl)
    return out.reshape(B, Lw, D)


# speedup vs baseline: 1.3361x; 1.3361x over previous
import functools
import jax
import jax.numpy as jnp
from jax import lax
from jax.experimental import pallas as pl
from jax.experimental.pallas import tpu as pltpu
from jax.experimental.pallas import tpu_sc as plsc

NC, NS, L = 2, 16, 16
NW = NC * NS
V, D = 1000, 128
DI = D // 2
NG = D // 32


def kernel(x_train_char, char_emb_weight):
    B, Lw, C = x_train_char.shape
    W = B * Lw
    wpb = W // NW
    CH = 200
    nchunk = wpb // CH

    wp = char_emb_weight.astype(jnp.bfloat16)
    wp = wp.reshape(V, NG, 2, 16).transpose(0, 1, 3, 2).reshape(V, DI, 2)
    tbl = lax.bitcast_convert_type(wp, jnp.int32).reshape(V * DI)

    idx = x_train_char.reshape(W * C)

    @functools.partial(
        pl.kernel,
        out_type=jax.ShapeDtypeStruct((W * D,), jnp.float32),
        mesh=plsc.VectorSubcoreMesh(
            core_axis_name="c", subcore_axis_name="s",
            num_cores=NC, num_subcores=NS,
        ),
        scratch_types=[
            pltpu.VMEM((V * DI,), jnp.int32),
            pltpu.VMEM((CH * C,), jnp.int32),
            pltpu.VMEM((CH * C,), jnp.int32),
            pltpu.VMEM((CH * D,), jnp.float32),
            pltpu.VMEM((CH * D,), jnp.float32),
            pltpu.SemaphoreType.DMA,
            pltpu.SemaphoreType.DMA,
            pltpu.SemaphoreType.DMA,
            pltpu.SemaphoreType.DMA,
        ],
        compiler_params=pltpu.CompilerParams(needs_layout_passes=False),
    )
    def sc_kernel(idx_hbm, tbl_hbm, out_hbm,
                  tbl_v, idx0, idx1, out0, out1,
                  isem0, isem1, osem0, osem1):
        wid = lax.axis_index("s") * NC + lax.axis_index("c")
        base_w = wid * wpb
        pltpu.async_copy(
            idx_hbm.at[pl.ds(base_w * C, CH * C)], idx0, isem0)
        pltpu.sync_copy(tbl_hbm, tbl_v)

        def do_word(idx_v, out_v, w):
            acc0 = [jnp.zeros((2 * L,), jnp.bfloat16) for _ in range(NG)]
            acc1 = [jnp.zeros((2 * L,), jnp.bfloat16) for _ in range(NG)]
            p = w * C
            iv0 = idx_v[pl.ds(p, L)]
            iv1 = idx_v[pl.ds(p + C - L, L)]
            for c in range(C):
                row = iv0[c] if c < L else iv1[c - (C - L)]
                rb = row * DI
                accs = acc0 if c % 2 == 0 else acc1
                for j in range(NG):
                    vals = tbl_v[pl.ds(rb + j * L, L)]
                    accs[j] = accs[j] + plsc.bitcast(vals, jnp.bfloat16)
            ob = w * D
            for j in range(NG):
                ai = plsc.bitcast(acc0[j] + acc1[j], jnp.int32)
                lo = plsc.bitcast(ai << 16, jnp.float32)
                hi = plsc.bitcast(ai & jnp.int32(-65536), jnp.float32)
                out_v[pl.ds(ob + 32 * j, L)] = lo
                out_v[pl.ds(ob + 32 * j + 16, L)] = hi

        def compute_chunk(idx_v, out_v):
            @plsc.parallel_loop(0, CH, unroll=4)
            def word_body(w):
                do_word(idx_v, out_v, w)

        def half(g, idx_v, out_v, isem_this, isem_next, osem, idx_next):
            pltpu.make_async_copy(
                idx_hbm.at[pl.ds(0, CH * C)], idx_v, isem_this).wait()
            gn = jnp.minimum(g + 1, nchunk - 1)
            pltpu.async_copy(
                idx_hbm.at[pl.ds((base_w + gn * CH) * C, CH * C)],
                idx_next, isem_next)

            @pl.when(g >= 2)
            def _():
                pltpu.make_async_copy(
                    out_v, out_hbm.at[pl.ds(0, CH * D)], osem).wait()

            compute_chunk(idx_v, out_v)
            pltpu.async_copy(
                out_v, out_hbm.at[pl.ds((base_w + g * CH) * D, CH * D)], osem)

        def body(g2, _):
            g = g2 * 2
            half(g, idx0, out0, isem0, isem1, osem0, idx1)
            half(g + 1, idx1, out1, isem1, isem0, osem1, idx0)
            return None

        lax.fori_loop(0, nchunk // 2, body, None)

        pltpu.make_async_copy(
            idx_hbm.at[pl.ds(0, CH * C)], idx0, isem0).wait()
        pltpu.make_async_copy(
            out0, out_hbm.at[pl.ds(0, CH * D)], osem0).wait()
        pltpu.make_async_copy(
            out1, out_hbm.at[pl.ds(0, CH * D)], osem1).wait()

    out = sc_kernel(idx, tbl)
    return out.reshape(B, Lw, D)
